# Initial kernel scaffold; baseline (speedup 1.0000x reference)
import jax
import jax.numpy as jnp
from jax.experimental import pallas as pl


def kernel(sim_matrix):
    raise NotImplementedError("write your pallas kernel here")


# speedup vs baseline: 6.6910x; 6.6910x over previous
import jax
import jax.numpy as jnp
from jax.experimental import pallas as pl

NQ = 20000
NT = 1000
NT_PAD = 1024
REL_POS_K = 5
REL_NEG_K = 10
NPAIR = REL_POS_K * NT
NPAIR_PAD = 5120
NQ_PAD = 20224
PAD_KEY = 1 << 29


def _topk_body(x_ref, idx_ref, key_ref):
    j = pl.program_id(0)
    x = x_ref[...]
    ii = jax.lax.broadcasted_iota(jnp.int32, (NQ, 128), 0)
    tcol = j * 128 + jax.lax.broadcasted_iota(jnp.int32, (16, 128), 1)

    m_prev = jnp.full((128,), 2.0, dtype=jnp.float32)
    i_prev = jnp.full((128,), -1, dtype=jnp.int32)
    idx_rows = []
    for _ in range(REL_NEG_K):
        rem = (x < m_prev[None, :]) | (
            (x == m_prev[None, :]) & (ii > i_prev[None, :]))
        vals = jnp.where(rem, x, -2.0)
        m = jnp.max(vals, axis=0)
        cand = jnp.where(rem & (x == m[None, :]), ii, NQ + 1)
        idx = jnp.min(cand, axis=0)
        idx_rows.append(idx)
        m_prev, i_prev = m, idx
    for _ in range(16 - REL_NEG_K):
        idx_rows.append(idx_rows[-1])
    top = jnp.stack(idx_rows, axis=0)
    idx_ref[...] = top
    key_ref[...] = (top << 10) | tcol


def _labels_body(ent_ref, lab_ref):
    qb = pl.program_id(0)
    qv = qb * 256 + jax.lax.broadcasted_iota(jnp.int32, (256, 512), 0)
    acc_pos = jnp.zeros((256,), dtype=jnp.bool_)
    acc_all = jnp.zeros((256,), dtype=jnp.bool_)
    for c in range(20):
        e_row = ent_ref[c, :]
        hit = jnp.any(qv == e_row[None, :], axis=1)
        if c < 10:
            acc_pos = acc_pos | hit
        acc_all = acc_all | hit
    lab = jnp.where(acc_pos, 1, jnp.where(acc_all, -1, 0)).astype(jnp.int32)
    lab_ref[...] = jnp.broadcast_to(lab[:, None], (256, 128))


def _rank_body(kb_ref, kl_ref, rank_ref):
    kb = kb_ref[...]
    acc = jnp.zeros((512,), dtype=jnp.int32)
    for c in range(40):
        k_row = kl_ref[c, :]
        acc = acc + jnp.sum((k_row[None, :] < kb).astype(jnp.int32), axis=1)
    rank_ref[...] = jnp.broadcast_to(acc[:, None], (512, 128))


def _permute_body(rl_ref, kl_ref, qry_ref, tgt_ref):
    pb = pl.program_id(0)
    pv = pb * 512 + jax.lax.broadcasted_iota(jnp.int32, (512, 128), 0)
    acc = jnp.zeros((512,), dtype=jnp.int32)
    for c in range(40):
        r_row = rl_ref[c, :]
        k_row = kl_ref[c, :]
        hit = pv == r_row[None, :]
        acc = acc + jnp.sum(jnp.where(hit, k_row[None, :], 0), axis=1)
    qry = acc >> 10
    tgt = acc & 1023
    qry_ref[...] = jnp.broadcast_to(qry[:, None], (512, 128))
    tgt_ref[...] = jnp.broadcast_to(tgt[:, None], (512, 128))


def kernel(sim_matrix):
    x = jnp.pad(sim_matrix, ((0, 0), (0, NT_PAD - NT)), constant_values=-1.0)

    top_idx, keys = pl.pallas_call(
        _topk_body,
        grid=(NT_PAD // 128,),
        in_specs=[pl.BlockSpec((NQ, 128), lambda j: (0, j))],
        out_specs=[pl.BlockSpec((16, 128), lambda j: (0, j)),
                   pl.BlockSpec((16, 128), lambda j: (0, j))],
        out_shape=[jax.ShapeDtypeStruct((16, NT_PAD), jnp.int32),
                   jax.ShapeDtypeStruct((16, NT_PAD), jnp.int32)],
    )(x)

    top10 = top_idx[:REL_NEG_K, :NT]
    pad1 = jnp.full((120,), -1, dtype=jnp.int32)
    entries = jnp.concatenate([
        top10[:REL_POS_K].ravel(), pad1,
        top10[REL_POS_K:].ravel(), pad1]).reshape(20, 512)

    pos_keys = jnp.concatenate([
        keys[:REL_POS_K, :NT].ravel(),
        jnp.full((NPAIR_PAD - NPAIR,), PAD_KEY, dtype=jnp.int32)])
    keys_bcast = jnp.broadcast_to(pos_keys[:, None], (NPAIR_PAD, 128))
    keys_lane = pos_keys.reshape(40, 128)

    labels = pl.pallas_call(
        _labels_body,
        grid=(NQ_PAD // 256,),
        in_specs=[pl.BlockSpec((20, 512), lambda q: (0, 0))],
        out_specs=pl.BlockSpec((256, 128), lambda q: (q, 0)),
        out_shape=jax.ShapeDtypeStruct((NQ_PAD, 128), jnp.int32),
    )(entries)

    ranks = pl.pallas_call(
        _rank_body,
        grid=(NPAIR_PAD // 512,),
        in_specs=[pl.BlockSpec((512, 128), lambda i: (i, 0)),
                  pl.BlockSpec((40, 128), lambda i: (0, 0))],
        out_specs=pl.BlockSpec((512, 128), lambda i: (i, 0)),
        out_shape=jax.ShapeDtypeStruct((NPAIR_PAD, 128), jnp.int32),
    )(keys_bcast, keys_lane)

    ranks_lane = ranks[:, 0].reshape(40, 128)

    qry_b, tgt_b = pl.pallas_call(
        _permute_body,
        grid=(NPAIR_PAD // 512,),
        in_specs=[pl.BlockSpec((40, 128), lambda p: (0, 0)),
                  pl.BlockSpec((40, 128), lambda p: (0, 0))],
        out_specs=[pl.BlockSpec((512, 128), lambda p: (p, 0)),
                   pl.BlockSpec((512, 128), lambda p: (p, 0))],
        out_shape=[jax.ShapeDtypeStruct((NPAIR_PAD, 128), jnp.int32),
                   jax.ShapeDtypeStruct((NPAIR_PAD, 128), jnp.int32)],
    )(ranks_lane, keys_lane)

    match_labels = labels[:NQ, 0]
    matched_qry_ids = qry_b[:NPAIR, 0]
    matched_tgt_ids = tgt_b[:NPAIR, 0]
    return (match_labels, matched_qry_ids, matched_tgt_ids)


# speedup vs baseline: 7.7979x; 1.1654x over previous
import jax
import jax.numpy as jnp
from jax.experimental import pallas as pl

NQ = 20000
NT = 1000
REL_POS_K = 5
REL_NEG_K = 10
NPAIR = REL_POS_K * NT
NPAIR_PAD = 5120
NQ_PAD = 20224
PAD_KEY = 1 << 29


def _topk_body(x_ref, ent_ref, key_ref):
    j = pl.program_id(0)
    x = x_ref[...]
    ii = jax.lax.broadcasted_iota(jnp.int32, (NQ, 128), 0)
    tcol = j * 128 + jax.lax.broadcasted_iota(jnp.int32, (1, 128), 1)[0]
    valid = tcol < NT

    m_prev = jnp.full((128,), 2.0, dtype=jnp.float32)
    i_prev = jnp.full((128,), -1, dtype=jnp.int32)
    for k in range(REL_NEG_K):
        rem = (x < m_prev[None, :]) | (
            (x == m_prev[None, :]) & (ii > i_prev[None, :]))
        vals = jnp.where(rem, x, -2.0)
        m = jnp.max(vals, axis=0)
        cand = jnp.where(rem & (x == m[None, :]), ii, NQ + 1)
        idx = jnp.min(cand, axis=0)
        ent_ref[pl.ds(k * 8 + j, 1), :] = jnp.where(valid, idx, -1)[None, :]
        if k < REL_POS_K:
            key = jnp.where(valid, (idx << 10) | tcol, PAD_KEY)
            key_ref[pl.ds(k * 8 + j, 1), :] = key[None, :]
        m_prev, i_prev = m, idx


def _labels_body(ent_ref, lab_ref):
    qb = pl.program_id(0)
    qv = qb * 256 + jax.lax.broadcasted_iota(jnp.int32, (256, 128), 0)
    acc_pos = jnp.zeros((256,), dtype=jnp.bool_)
    acc_all = jnp.zeros((256,), dtype=jnp.bool_)
    for c in range(80):
        e_row = ent_ref[c, :]
        hit = jnp.any(qv == e_row[None, :], axis=1)
        if c < 40:
            acc_pos = acc_pos | hit
        acc_all = acc_all | hit
    lab = jnp.where(acc_pos, 1, jnp.where(acc_all, -1, 0)).astype(jnp.int32)
    lab_ref[...] = jnp.broadcast_to(lab[:, None], (256, 8))


def _rank_body(kb_ref, kl_ref, rank_ref):
    kb = kb_ref[...]
    acc = jnp.zeros((512,), dtype=jnp.int32)
    for c in range(40):
        k_row = kl_ref[c, :]
        acc = acc + jnp.sum((k_row[None, :] < kb).astype(jnp.int32), axis=1)
    rank_ref[...] = jnp.broadcast_to(acc[:, None], (512, 8))


def _permute_body(rl_ref, kl_ref, qry_ref, tgt_ref):
    pb = pl.program_id(0)
    pv = pb * 512 + jax.lax.broadcasted_iota(jnp.int32, (512, 128), 0)
    acc = jnp.zeros((512,), dtype=jnp.int32)
    for c in range(40):
        r_row = rl_ref[c, :]
        k_row = kl_ref[c, :]
        hit = pv == r_row[None, :]
        acc = acc + jnp.sum(jnp.where(hit, k_row[None, :], 0), axis=1)
    qry = acc >> 10
    tgt = acc & 1023
    qry_ref[...] = jnp.broadcast_to(qry[:, None], (512, 8))
    tgt_ref[...] = jnp.broadcast_to(tgt[:, None], (512, 8))


def kernel(sim_matrix):
    ent, keys_lane = pl.pallas_call(
        _topk_body,
        grid=(8,),
        in_specs=[pl.BlockSpec((NQ, 128), lambda j: (0, j))],
        out_specs=[pl.BlockSpec((80, 128), lambda j: (0, 0)),
                   pl.BlockSpec((40, 128), lambda j: (0, 0))],
        out_shape=[jax.ShapeDtypeStruct((80, 128), jnp.int32),
                   jax.ShapeDtypeStruct((40, 128), jnp.int32)],
    )(sim_matrix)

    keys_bcast = jnp.broadcast_to(
        keys_lane.ravel()[:, None], (NPAIR_PAD, 128))

    labels = pl.pallas_call(
        _labels_body,
        grid=(NQ_PAD // 256,),
        in_specs=[pl.BlockSpec((80, 128), lambda q: (0, 0))],
        out_specs=pl.BlockSpec((256, 8), lambda q: (q, 0)),
        out_shape=jax.ShapeDtypeStruct((NQ_PAD, 8), jnp.int32),
    )(ent)

    ranks = pl.pallas_call(
        _rank_body,
        grid=(NPAIR_PAD // 512,),
        in_specs=[pl.BlockSpec((512, 128), lambda i: (i, 0)),
                  pl.BlockSpec((40, 128), lambda i: (0, 0))],
        out_specs=pl.BlockSpec((512, 8), lambda i: (i, 0)),
        out_shape=jax.ShapeDtypeStruct((NPAIR_PAD, 8), jnp.int32),
    )(keys_bcast, keys_lane)

    ranks_lane = ranks[:, 0].reshape(40, 128)

    qry_b, tgt_b = pl.pallas_call(
        _permute_body,
        grid=(NPAIR_PAD // 512,),
        in_specs=[pl.BlockSpec((40, 128), lambda p: (0, 0)),
                  pl.BlockSpec((40, 128), lambda p: (0, 0))],
        out_specs=[pl.BlockSpec((512, 8), lambda p: (p, 0)),
                   pl.BlockSpec((512, 8), lambda p: (p, 0))],
        out_shape=[jax.ShapeDtypeStruct((NPAIR_PAD, 8), jnp.int32),
                   jax.ShapeDtypeStruct((NPAIR_PAD, 8), jnp.int32)],
    )(ranks_lane, keys_lane)

    match_labels = labels[:NQ, 0]
    matched_qry_ids = qry_b[:NPAIR, 0]
    matched_tgt_ids = tgt_b[:NPAIR, 0]
    return (match_labels, matched_qry_ids, matched_tgt_ids)


# speedup vs baseline: 7.8247x; 1.0034x over previous
import jax
import jax.numpy as jnp
from jax.experimental import pallas as pl

NQ = 20000
NT = 1000
REL_POS_K = 5
REL_NEG_K = 10
NPAIR = REL_POS_K * NT
NPAIR_PAD = 5120
NQ_PAD = 20224
PAD_KEY = 1 << 29


def _topk_body(x_ref, ent_ref, key_ref):
    j = pl.program_id(0)
    x = x_ref[...]
    ii = jax.lax.broadcasted_iota(jnp.int32, (NQ, 128), 0)
    tcol = j * 128 + jax.lax.broadcasted_iota(jnp.int32, (1, 128), 1)[0]
    valid = tcol < NT

    m_prev = jnp.full((128,), 2.0, dtype=jnp.float32)
    i_prev = jnp.full((128,), -1, dtype=jnp.int32)
    for k in range(REL_NEG_K):
        if k == 0:
            m = jnp.max(x, axis=0)
            cand = jnp.where(x == m[None, :], ii, NQ + 1)
        else:
            rem = (x < m_prev[None, :]) | (
                (x == m_prev[None, :]) & (ii > i_prev[None, :]))
            vals = jnp.where(rem, x, -2.0)
            m = jnp.max(vals, axis=0)
            cand = jnp.where(rem & (x == m[None, :]), ii, NQ + 1)
        idx = jnp.min(cand, axis=0)
        ent_ref[pl.ds(k * 8 + j, 1), :] = jnp.where(valid, idx, -1)[None, :]
        if k < REL_POS_K:
            key = jnp.where(valid, (idx << 10) | tcol, PAD_KEY)
            key_ref[pl.ds(k * 8 + j, 1), :] = key[None, :]
        m_prev, i_prev = m, idx


def _labels_body(ent_ref, lab_ref):
    qb = pl.program_id(0)
    qv = qb * 256 + jax.lax.broadcasted_iota(jnp.int32, (256, 128), 0)
    acc_pos = jnp.zeros((256,), dtype=jnp.bool_)
    acc_all = jnp.zeros((256,), dtype=jnp.bool_)
    for c in range(80):
        e_row = ent_ref[c, :]
        hit = jnp.any(qv == e_row[None, :], axis=1)
        if c < 40:
            acc_pos = acc_pos | hit
        acc_all = acc_all | hit
    lab = jnp.where(acc_pos, 1, jnp.where(acc_all, -1, 0)).astype(jnp.int32)
    lab_ref[...] = jnp.broadcast_to(lab[:, None], (256, 8))


def _rank_body(kb_ref, kl_ref, rank_ref):
    kb = kb_ref[...]
    acc = jnp.zeros((512,), dtype=jnp.int32)
    for c in range(40):
        k_row = kl_ref[c, :]
        acc = acc + jnp.sum((k_row[None, :] < kb).astype(jnp.int32), axis=1)
    rank_ref[...] = jnp.broadcast_to(acc[:, None], (512, 8))


def _permute_body(rl_ref, kl_ref, qry_ref, tgt_ref):
    pb = pl.program_id(0)
    pv = pb * 512 + jax.lax.broadcasted_iota(jnp.int32, (512, 128), 0)
    acc = jnp.zeros((512,), dtype=jnp.int32)
    for c in range(40):
        r_row = rl_ref[c, :]
        k_row = kl_ref[c, :]
        hit = pv == r_row[None, :]
        acc = acc + jnp.sum(jnp.where(hit, k_row[None, :], 0), axis=1)
    qry = acc >> 10
    tgt = acc & 1023
    qry_ref[...] = jnp.broadcast_to(qry[:, None], (512, 8))
    tgt_ref[...] = jnp.broadcast_to(tgt[:, None], (512, 8))


def kernel(sim_matrix):
    ent, keys_lane = pl.pallas_call(
        _topk_body,
        grid=(8,),
        in_specs=[pl.BlockSpec((NQ, 128), lambda j: (0, j))],
        out_specs=[pl.BlockSpec((80, 128), lambda j: (0, 0)),
                   pl.BlockSpec((40, 128), lambda j: (0, 0))],
        out_shape=[jax.ShapeDtypeStruct((80, 128), jnp.int32),
                   jax.ShapeDtypeStruct((40, 128), jnp.int32)],
    )(sim_matrix)

    keys_bcast = jnp.broadcast_to(
        keys_lane.ravel()[:, None], (NPAIR_PAD, 128))

    labels = pl.pallas_call(
        _labels_body,
        grid=(NQ_PAD // 256,),
        in_specs=[pl.BlockSpec((80, 128), lambda q: (0, 0))],
        out_specs=pl.BlockSpec((256, 8), lambda q: (q, 0)),
        out_shape=jax.ShapeDtypeStruct((NQ_PAD, 8), jnp.int32),
    )(ent)

    ranks = pl.pallas_call(
        _rank_body,
        grid=(NPAIR_PAD // 512,),
        in_specs=[pl.BlockSpec((512, 128), lambda i: (i, 0)),
                  pl.BlockSpec((40, 128), lambda i: (0, 0))],
        out_specs=pl.BlockSpec((512, 8), lambda i: (i, 0)),
        out_shape=jax.ShapeDtypeStruct((NPAIR_PAD, 8), jnp.int32),
    )(keys_bcast, keys_lane)

    ranks_lane = ranks[:, 0].reshape(40, 128)

    qry_b, tgt_b = pl.pallas_call(
        _permute_body,
        grid=(NPAIR_PAD // 512,),
        in_specs=[pl.BlockSpec((40, 128), lambda p: (0, 0)),
                  pl.BlockSpec((40, 128), lambda p: (0, 0))],
        out_specs=[pl.BlockSpec((512, 8), lambda p: (p, 0)),
                   pl.BlockSpec((512, 8), lambda p: (p, 0))],
        out_shape=[jax.ShapeDtypeStruct((NPAIR_PAD, 8), jnp.int32),
                   jax.ShapeDtypeStruct((NPAIR_PAD, 8), jnp.int32)],
    )(ranks_lane, keys_lane)

    match_labels = labels[:NQ, 0]
    matched_qry_ids = qry_b[:NPAIR, 0]
    matched_tgt_ids = tgt_b[:NPAIR, 0]
    return (match_labels, matched_qry_ids, matched_tgt_ids)
